# baseline (speedup 1.0000x reference)
import functools

import jax
import jax.numpy as jnp
from jax import lax
from jax.experimental import pallas as pl
from jax.experimental.pallas import tpu as pltpu
from jax.experimental.pallas import tpu_sc as plsc

B = 16384
D = 64
NW = 32
BPW = B // NW
ICH = 128
NCH = BPW // ICH
L = 16


def _score_kernel(user_hbm, item_hbm, rel_hbm, hidx_hbm, ridx_hbm, tidx_hbm,
                  out_hbm, hidx_v, ridx_v, tidx_v, h_v, r_v, t_v, s_v,
                  sem_h, sem_r, sem_t):
    wid = lax.axis_index("s") * 2 + lax.axis_index("c")
    base = wid * BPW

    pltpu.sync_copy(hidx_hbm.at[pl.ds(wid * NCH, NCH)], hidx_v)
    pltpu.sync_copy(ridx_hbm.at[pl.ds(wid * NCH, NCH)], ridx_v)
    pltpu.sync_copy(tidx_hbm.at[pl.ds(wid * NCH, NCH)], tidx_v)

    copies = []
    for j in range(NCH):
        dst = pl.ds(j * ICH, ICH)
        copies.append(pltpu.async_copy(user_hbm.at[hidx_v.at[j]], h_v.at[dst], sem_h))
        copies.append(pltpu.async_copy(rel_hbm.at[ridx_v.at[j]], r_v.at[dst], sem_r))
        copies.append(pltpu.async_copy(item_hbm.at[tidx_v.at[j]], t_v.at[dst], sem_t))
    for c in copies:
        c.wait()

    lanes = lax.iota(jnp.int32, L)
    lane0 = lanes == 0
    perms = [lanes ^ s for s in (8, 4, 2, 1)]

    def row(i, _):
        acc = jnp.zeros((L,), jnp.float32)
        for c in range(D // L):
            sl = pl.ds(c * L, L)
            acc = acc + jnp.abs(h_v[i, sl] + r_v[i, sl] - t_v[i, sl])
        for p in perms:
            acc = acc + acc.at[p].get(mode="promise_in_bounds", unique_indices=True)
        plsc.store_scatter(s_v, [jnp.broadcast_to(i, (L,))], -acc, mask=lane0)
        return 0

    lax.fori_loop(0, BPW, row, 0)

    pltpu.sync_copy(s_v, out_hbm.at[pl.ds(base, BPW)])


@jax.jit
def _score(user_table, item_table, rel_table, head_idx, relation_idx, tail_idx):
    mesh = plsc.VectorSubcoreMesh(core_axis_name="c", subcore_axis_name="s")
    kern = functools.partial(
        pl.kernel,
        mesh=mesh,
        compiler_params=pltpu.CompilerParams(
            needs_layout_passes=False, use_tc_tiling_on_sc=False),
        out_type=jax.ShapeDtypeStruct((B,), jnp.float32),
        scratch_types=[
            pltpu.VMEM((NCH, ICH), jnp.int32),
            pltpu.VMEM((NCH, ICH), jnp.int32),
            pltpu.VMEM((NCH, ICH), jnp.int32),
            pltpu.VMEM((BPW, D), jnp.float32),
            pltpu.VMEM((BPW, D), jnp.float32),
            pltpu.VMEM((BPW, D), jnp.float32),
            pltpu.VMEM((BPW,), jnp.float32),
            pltpu.SemaphoreType.DMA,
            pltpu.SemaphoreType.DMA,
            pltpu.SemaphoreType.DMA,
        ],
    )(_score_kernel)
    return kern(user_table, item_table, rel_table,
                head_idx.reshape(NW * NCH, ICH),
                relation_idx.reshape(NW * NCH, ICH),
                tail_idx.reshape(NW * NCH, ICH))


def kernel(user_table, item_table, rel_table, head_idx, relation_idx, tail_idx):
    return _score(user_table, item_table, rel_table,
                  head_idx.astype(jnp.int32),
                  relation_idx.astype(jnp.int32),
                  tail_idx.astype(jnp.int32))
